# Initial kernel scaffold; baseline (speedup 1.0000x reference)
import jax
import jax.numpy as jnp
from jax.experimental import pallas as pl


def kernel(h0, b0, h1, b1, s1_W0, s1_b0, s2_W0, s2_b0, proj_W0, proj_b0, s1_W1, s1_b1, s2_W1, s2_b1, proj_W1, proj_b1, ln_g, ln_b, f1_W, f1_b, f2_W, f2_b):
    raise NotImplementedError("write your pallas kernel here")


# speedup vs baseline: 7.2400x; 7.2400x over previous
import functools

import jax
import jax.numpy as jnp
from jax import lax
from jax.experimental import pallas as pl
from jax.experimental.pallas import tpu as pltpu
from jax.experimental.pallas import tpu_sc as plsc

H = 128
G = 512
NW = 32
CHUNK = 3128
T = 128
NEG = -3.0e38


def _score_body(h_ref, w1_ref, b1_ref, w2_ref, e_ref):
    u = jnp.tanh(
        jnp.dot(h_ref[...], w1_ref[...], preferred_element_type=jnp.float32)
        + b1_ref[...][None, :])
    s = jnp.sum(u * w2_ref[...][None, :], axis=1)
    e_ref[...] = jnp.exp(s)


def _scores(h, s1W, s1b, s2W, s2b):
    n = h.shape[0]
    blk = 512
    grid = (n + blk - 1) // blk
    return pl.pallas_call(
        _score_body,
        grid=(grid,),
        in_specs=[
            pl.BlockSpec((blk, H), lambda i: (i, 0)),
            pl.BlockSpec((H, H), lambda i: (0, 0)),
            pl.BlockSpec((H,), lambda i: (0,)),
            pl.BlockSpec((H,), lambda i: (0,)),
        ],
        out_specs=pl.BlockSpec((blk,), lambda i: (i,)),
        out_shape=jax.ShapeDtypeStruct((n,), jnp.float32),
    )(h, s1W, s1b, s2W[:, 0])


def _sc_pool(h, b, e):
    n = h.shape[0]
    mesh = plsc.VectorSubcoreMesh(core_axis_name="c", subcore_axis_name="s")
    ROW = 4 * H

    @functools.partial(
        pl.kernel,
        mesh=mesh,
        out_type=[
            jax.ShapeDtypeStruct((G * ROW,), jnp.float32),
            jax.ShapeDtypeStruct((NW * ROW,), jnp.float32),
        ],
        scratch_types=[
            pltpu.VMEM((T * H,), jnp.float32),
            pltpu.VMEM((T + 16,), jnp.int32),
            pltpu.VMEM((T + 16,), jnp.float32),
            pltpu.VMEM((ROW,), jnp.float32),
            pltpu.VMEM((ROW,), jnp.float32),
            pltpu.VMEM((16,), jnp.float32),
            pltpu.VMEM((16,), jnp.int32),
            pltpu.VMEM((16,), jnp.int32),
        ],
    )
    def kfn(h_hbm, b_hbm, e_hbm, tab_hbm, head_hbm,
            hbuf, bbuf, ebuf, stage, zrow, hinit, bprev, bfirst):
        cid = lax.axis_index("c")
        sid = lax.axis_index("s")
        w = sid * 2 + cid
        r0 = w * CHUNK
        clen = jnp.where(w == NW - 1, n - (NW - 1) * CHUNK, CHUNK)
        lanes = lax.iota(jnp.int32, 16)
        zvec = jnp.zeros((16,), jnp.float32)

        for j in range(ROW // 16):
            zrow[pl.ds(16 * j, 16)] = zvec
        for j in range(1, 8):
            stage[pl.ds(3 * H + 16 * j, 16)] = zvec
        hinit[...] = jnp.where(lanes == 2, -1.0, 0.0)

        pltpu.sync_copy(zrow, head_hbm.at[pl.ds(w * ROW, ROW)])
        pltpu.sync_copy(hinit, head_hbm.at[pl.ds(w * ROW + 3 * H, 16)])

        @pl.when(w > 0)
        def _():
            pltpu.sync_copy(b_hbm.at[pl.ds(r0 - 8, 8)], bprev.at[pl.ds(0, 8)])
        pltpu.sync_copy(b_hbm.at[pl.ds(r0, 8)], bfirst.at[pl.ds(0, 8)])
        s_own = jnp.where(w == 0, 0, bprev[pl.ds(0, 16)][7] + 1)

        def zero_seg(g, _):
            pltpu.sync_copy(zrow, tab_hbm.at[pl.ds(g * ROW, ROW)])
            return 0

        def emit_flush(cur_g, nextzero, den, cnt, accs):
            for j in range(8):
                stage[pl.ds(16 * j, 16)] = accs[j]
            for j in range(8):
                stage[pl.ds(H + 16 * j, 16)] = accs[8 + j]
            for j in range(8):
                stage[pl.ds(2 * H + 16 * j, 16)] = accs[16 + j]
            gf = cur_g.astype(jnp.float32)
            aux = (jnp.where(lanes == 0, den, 0.0)
                   + jnp.where(lanes == 1, cnt, 0.0)
                   + jnp.where(lanes == 2, gf, 0.0))
            stage[pl.ds(3 * H, 16)] = aux

            @pl.when(cur_g < s_own)
            def _():
                pltpu.sync_copy(stage, head_hbm.at[pl.ds(w * ROW, ROW)])

            @pl.when(cur_g >= s_own)
            def _():
                lax.fori_loop(nextzero, cur_g, zero_seg, 0)
                pltpu.sync_copy(stage, tab_hbm.at[pl.ds(cur_g * ROW, ROW)])

        zaccs = tuple(jnp.zeros((16,), jnp.float32) for _ in range(16)) + \
            tuple(jnp.full((16,), NEG, jnp.float32) for _ in range(8))

        def row_body(i, carry):
            cur_g, nextzero, den, cnt = carry[:4]
            accs = list(carry[4:])
            g = bbuf[pl.ds(i, 16)][0]
            changed = g != cur_g

            @pl.when(changed)
            def _():
                emit_flush(cur_g, nextzero, den, cnt, accs)

            nextzero = jnp.where(changed & (cur_g >= s_own),
                                 cur_g + 1, nextzero)
            den = jnp.where(changed, 0.0, den)
            cnt = jnp.where(changed, 0.0, cnt)
            ev = ebuf[pl.ds(i, 16)][0]
            for j in range(8):
                v = hbuf[pl.ds(i * H + 16 * j, 16)]
                accs[j] = jnp.where(changed, 0.0, accs[j]) + v
                accs[8 + j] = jnp.where(changed, 0.0, accs[8 + j]) + ev * v
                accs[16 + j] = jnp.maximum(
                    jnp.where(changed, NEG, accs[16 + j]), v)
            return (g, nextzero, den + ev, cnt + 1.0) + tuple(accs)

        def tile_body(base, i0, carry):
            pltpu.sync_copy(h_hbm.at[pl.ds(base * H, T * H)], hbuf)
            pltpu.sync_copy(b_hbm.at[pl.ds(base, T)], bbuf.at[pl.ds(0, T)])
            pltpu.sync_copy(e_hbm.at[pl.ds(base, T)], ebuf.at[pl.ds(0, T)])
            return lax.fori_loop(i0, T, row_body, carry)

        carry0 = (bfirst[pl.ds(0, 16)][0], s_own, 0.0, 0.0) + zaccs
        ntiles = clen // T
        rem = clen - ntiles * T

        def full_tile(t, carry):
            return tile_body(r0 + t * T, 0, carry)

        carry1 = lax.fori_loop(0, ntiles, full_tile, carry0)
        carry2 = tile_body(r0 + clen - T, T - rem, carry1)

        cur_g, nextzero, den, cnt = carry2[:4]
        emit_flush(cur_g, nextzero, den, cnt, list(carry2[4:]))
        nz = jnp.where(cur_g >= s_own, cur_g + 1, nextzero)
        upper = jnp.where(w == NW - 1, G, cur_g + 1)
        lax.fori_loop(nz, upper, zero_seg, 0)

    tab, head = kfn(h.reshape(-1), b, e)
    return tab.reshape(G, 4, H), head.reshape(NW, 4, H)


def _merge_body(tab0_ref, head0_ref, tab1_ref, head1_ref,
                pw0_ref, pb0_ref, pw1_ref, pb1_ref,
                lng_ref, lnb_ref, f1w_ref, f1b_ref, f2w_ref, f2b_ref,
                out_ref):
    iota_g = lax.broadcasted_iota(jnp.int32, (G, 1), 0)
    lane2 = (lax.broadcasted_iota(jnp.int32, (1, H), 1) < 2)\
        .astype(jnp.float32)

    def branch(tab_ref, head_ref, pw_ref, pb_ref):
        tab = tab_ref[...]
        s_p = tab[:, 0, :]
        a_n = tab[:, 1, :]
        m_p = tab[:, 2, :]
        aux = tab[:, 3, :]
        head = head_ref[...]
        for w in range(NW):
            idf = head[w, 3, 2]
            valid = idf >= 0.0
            mask = (iota_g == idf.astype(jnp.int32)) & valid
            maskf = mask.astype(jnp.float32)
            s_p = s_p + maskf * head[w, 0, :][None, :]
            a_n = a_n + maskf * head[w, 1, :][None, :]
            m_p = jnp.maximum(m_p, jnp.where(mask, head[w, 2, :][None, :],
                                             NEG))
            aux = aux + maskf * (head[w, 3, :][None, :] * lane2)
        den = aux[:, 0:1]
        cnt = aux[:, 1:2]
        mean = s_p / jnp.maximum(cnt, 1.0)
        att = a_n / jnp.maximum(den, 1e-30)
        agg = jnp.concatenate([s_p, mean, m_p, att], axis=1)
        return (jnp.dot(agg, pw_ref[...], preferred_element_type=jnp.float32)
                + pb_ref[...][None, :])

    v0 = branch(tab0_ref, head0_ref, pw0_ref, pb0_ref)
    v1 = branch(tab1_ref, head1_ref, pw1_ref, pb1_ref)
    state = jnp.concatenate([v0, v1], axis=1)
    mu = jnp.mean(state, axis=-1, keepdims=True)
    var = jnp.mean((state - mu) ** 2, axis=-1, keepdims=True)
    x = (state - mu) * lax.rsqrt(var + 1e-5) * lng_ref[...][None, :] \
        + lnb_ref[...][None, :]
    x = x * jax.nn.sigmoid(x)
    x = jnp.dot(x, f1w_ref[...], preferred_element_type=jnp.float32) \
        + f1b_ref[...][None, :]
    x = x * jax.nn.sigmoid(x)
    out_ref[...] = jnp.dot(x, f2w_ref[...],
                           preferred_element_type=jnp.float32) \
        + f2b_ref[...][None, :]


def _merge(tab0, head0, tab1, head1, pW0, pb0, pW1, pb1,
           ln_g, ln_b, f1W, f1b, f2W, f2b):
    return pl.pallas_call(
        _merge_body,
        out_shape=jax.ShapeDtypeStruct((G, 8), jnp.float32),
    )(tab0, head0, tab1, head1, pW0, pb0, pW1, pb1,
      ln_g, ln_b, f1W, f1b, f2W, f2b)


def kernel(h0, b0, h1, b1, s1_W0, s1_b0, s2_W0, s2_b0, proj_W0, proj_b0,
           s1_W1, s1_b1, s2_W1, s2_b1, proj_W1, proj_b1,
           ln_g, ln_b, f1_W, f1_b, f2_W, f2_b):
    e0 = _scores(h0, s1_W0, s1_b0, s2_W0, s2_b0)
    e1 = _scores(h1, s1_W1, s1_b1, s2_W1, s2_b1)
    tab0, head0 = _sc_pool(h0, b0, e0)
    tab1, head1 = _sc_pool(h1, b1, e1)
    return _merge(tab0, head0, tab1, head1, proj_W0, proj_b0,
                  proj_W1, proj_b1, ln_g, ln_b, f1_W, f1_b, f2_W, f2_b)


# speedup vs baseline: 13.8635x; 1.9148x over previous
import functools

import jax
import jax.numpy as jnp
from jax import lax
from jax.experimental import pallas as pl
from jax.experimental.pallas import tpu as pltpu
from jax.experimental.pallas import tpu_sc as plsc

H = 128
G = 512
NW = 32
CHUNK = 3128
T = 128
NEG = -3.0e38


def _score_body(h_ref, w1_ref, b1_ref, w2_ref, e_ref):
    u = jnp.tanh(
        jnp.dot(h_ref[...], w1_ref[...], preferred_element_type=jnp.float32)
        + b1_ref[...][None, :])
    s = jnp.sum(u * w2_ref[...][None, :], axis=1)
    e_ref[...] = jnp.exp(s)


def _scores(h, s1W, s1b, s2W, s2b):
    n = h.shape[0]
    blk = 2048
    grid = (n + blk - 1) // blk
    return pl.pallas_call(
        _score_body,
        grid=(grid,),
        in_specs=[
            pl.BlockSpec((blk, H), lambda i: (i, 0)),
            pl.BlockSpec((H, H), lambda i: (0, 0)),
            pl.BlockSpec((H,), lambda i: (0,)),
            pl.BlockSpec((H,), lambda i: (0,)),
        ],
        out_specs=pl.BlockSpec((blk,), lambda i: (i,)),
        out_shape=jax.ShapeDtypeStruct((n,), jnp.float32),
    )(h, s1W, s1b, s2W[:, 0])


def _sc_pool(h, b, e):
    n = h.shape[0]
    mesh = plsc.VectorSubcoreMesh(core_axis_name="c", subcore_axis_name="s")
    ROW = 4 * H
    TT = 480
    NTF = 6
    NGRP = TT // 16
    assert CHUNK // TT == NTF and (n - (NW - 1) * CHUNK) // TT == NTF

    @functools.partial(
        pl.kernel,
        mesh=mesh,
        out_type=[
            jax.ShapeDtypeStruct((G * ROW,), jnp.float32),
            jax.ShapeDtypeStruct((NW * ROW,), jnp.float32),
        ],
        scratch_types=[
            pltpu.VMEM((TT * H,), jnp.float32),
            pltpu.VMEM((TT * H,), jnp.float32),
            pltpu.VMEM((TT + 16,), jnp.int32),
            pltpu.VMEM((TT + 16,), jnp.int32),
            pltpu.VMEM((TT + 16,), jnp.float32),
            pltpu.VMEM((TT + 16,), jnp.float32),
            pltpu.VMEM((ROW,), jnp.float32),
            pltpu.VMEM((ROW,), jnp.float32),
            pltpu.VMEM((16,), jnp.float32),
            pltpu.VMEM((16,), jnp.int32),
            pltpu.VMEM((16,), jnp.int32),
            pltpu.SMEM((2,), jnp.int32),
            pltpu.SemaphoreType.DMA,
            pltpu.SemaphoreType.DMA,
        ],
    )
    def kfn(h_hbm, b_hbm, e_hbm, tab_hbm, head_hbm,
            hb0, hb1, bb0, bb1, eb0, eb1, acc, zrow, hinit,
            bprev, bfirst, st, sem0, sem1):
        cid = lax.axis_index("c")
        sid = lax.axis_index("s")
        w = sid * 2 + cid
        r0 = w * CHUNK
        clen = jnp.where(w == NW - 1, n - (NW - 1) * CHUNK, CHUNK)
        rem = clen - NTF * TT
        lanes = lax.iota(jnp.int32, 16)
        zvec = jnp.zeros((16,), jnp.float32)
        nvec = jnp.full((16,), NEG, jnp.float32)
        onev = jnp.ones((16,), jnp.float32)

        for j in range(ROW // 16):
            zrow[pl.ds(16 * j, 16)] = zvec
            acc[pl.ds(16 * j, 16)] = nvec if 16 <= j < 24 else zvec
        hinit[...] = jnp.where(lanes == 0, -1.0, 0.0)

        pltpu.sync_copy(zrow, head_hbm.at[pl.ds(w * ROW, ROW)])
        pltpu.sync_copy(hinit, head_hbm.at[pl.ds(w * ROW + 3 * H + 32, 16)])

        @pl.when(w > 0)
        def _():
            pltpu.sync_copy(b_hbm.at[pl.ds(r0 - 8, 8)], bprev.at[pl.ds(0, 8)])
        pltpu.sync_copy(b_hbm.at[pl.ds(r0, 8)], bfirst.at[pl.ds(0, 8)])
        s_own = jnp.where(w == 0, 0, bprev[pl.ds(0, 16)][7] + 1)
        st[0] = bfirst[pl.ds(0, 16)][0]
        st[1] = s_own

        def base_of(t):
            return jnp.where(t == NTF, r0 + clen - TT, r0 + t * TT)

        bufs = [(hb0, bb0, eb0, sem0), (hb1, bb1, eb1, sem1)]

        def tile_copies(t, buf):
            hb, bb, eb, sem = buf
            base = base_of(t)
            return (
                pltpu.make_async_copy(
                    h_hbm.at[pl.ds(base * H, TT * H)], hb, sem),
                pltpu.make_async_copy(
                    b_hbm.at[pl.ds(base, TT)], bb.at[pl.ds(0, TT)], sem),
                pltpu.make_async_copy(
                    e_hbm.at[pl.ds(base, TT)], eb.at[pl.ds(0, TT)], sem),
            )

        def zero_seg(gg, c):
            pltpu.sync_copy(zrow, tab_hbm.at[pl.ds(gg * ROW, ROW)])
            return c

        def emit_flush():
            cur_g = st[0]
            nz = st[1]
            acc[pl.ds(3 * H + 32, 16)] = jnp.where(
                lanes == 0, cur_g.astype(jnp.float32), 0.0)

            @pl.when(cur_g < s_own)
            def _():
                pltpu.sync_copy(acc, head_hbm.at[pl.ds(w * ROW, ROW)])

            @pl.when(cur_g >= s_own)
            def _():
                lax.fori_loop(nz, cur_g, zero_seg, 0)
                pltpu.sync_copy(acc, tab_hbm.at[pl.ds(cur_g * ROW, ROW)])

            st[1] = jnp.where(cur_g >= s_own, cur_g + 1, nz)
            for j in range(8):
                acc[pl.ds(16 * j, 16)] = zvec
                acc[pl.ds(H + 16 * j, 16)] = zvec
                acc[pl.ds(2 * H + 16 * j, 16)] = nvec
            acc[pl.ds(3 * H, 16)] = zvec
            acc[pl.ds(3 * H + 16, 16)] = zvec

        def row_slow(i, buf):
            hb, bb, eb, _ = buf
            g = bb[pl.ds(i, 16)][0]

            @pl.when(g != st[0])
            def _():
                emit_flush()

            st[0] = g
            ev = eb[pl.ds(i, 16)][0]
            plsc.addupdate(acc.at[pl.ds(3 * H, 16)],
                           jnp.where(lanes == 0, ev, 0.0))
            plsc.addupdate(acc.at[pl.ds(3 * H + 16, 16)],
                           jnp.where(lanes == 0, 1.0, 0.0))
            for j in range(8):
                v = hb[pl.ds(i * H + 16 * j, 16)]
                plsc.addupdate(acc.at[pl.ds(16 * j, 16)], v)
                plsc.addupdate(acc.at[pl.ds(H + 16 * j, 16)], ev * v)
                mj = acc[pl.ds(2 * H + 16 * j, 16)]
                acc[pl.ds(2 * H + 16 * j, 16)] = jnp.maximum(mj, v)

        def group_fast(i, buf):
            hb, _, eb, _ = buf
            evec = eb[pl.ds(i, 16)]
            plsc.addupdate(acc.at[pl.ds(3 * H, 16)], evec)
            plsc.addupdate(acc.at[pl.ds(3 * H + 16, 16)], onev)
            s_r = [acc[pl.ds(16 * j, 16)] for j in range(8)]
            a_r = [acc[pl.ds(H + 16 * j, 16)] for j in range(8)]
            m_r = [acc[pl.ds(2 * H + 16 * j, 16)] for j in range(8)]

            def quad(q, carry):
                regs = list(carry)
                ebase = eb[pl.ds(i + 4 * q, 16)]
                for r in range(4):
                    ev = ebase[r]
                    for j in range(8):
                        v = hb[pl.ds((i + 4 * q + r) * H + 16 * j, 16)]
                        regs[j] = regs[j] + v
                        regs[8 + j] = regs[8 + j] + ev * v
                        regs[16 + j] = jnp.maximum(regs[16 + j], v)
                return tuple(regs)

            out = lax.fori_loop(0, 4, quad, tuple(s_r + a_r + m_r))
            for j in range(8):
                acc[pl.ds(16 * j, 16)] = out[j]
                acc[pl.ds(H + 16 * j, 16)] = out[8 + j]
                acc[pl.ds(2 * H + 16 * j, 16)] = out[16 + j]

        def process_tile(buf, is_tail):
            if is_tail:
                i0 = TT - rem
                lead = (16 - (i0 % 16)) % 16

                def lead_body(i, c):
                    row_slow(i, buf)
                    return c

                lax.fori_loop(i0, i0 + lead, lead_body, 0)
                g0 = (i0 + lead) // 16
            else:
                g0 = 0

            def group_body(gi, c):
                i = 16 * gi
                bvec = buf[1][pl.ds(i, 16)]
                fast = (bvec[0] == st[0]) & (bvec[15] == st[0])

                @pl.when(fast)
                def _():
                    group_fast(i, buf)

                @pl.when(jnp.logical_not(fast))
                def _():
                    def rb(r, cc):
                        row_slow(i + r, buf)
                        return cc
                    lax.fori_loop(0, 16, rb, 0)
                return c

            lax.fori_loop(g0, NGRP, group_body, 0)

        for c in tile_copies(0, bufs[0]):
            c.start()

        def do_tile(t, bi, is_tail):
            for cpy in tile_copies(t, bufs[bi]):
                cpy.wait()
            if not is_tail:
                for cpy in tile_copies(t + 1, bufs[1 - bi]):
                    cpy.start()
            process_tile(bufs[bi], is_tail)

        def pair_body(t2, c):
            do_tile(2 * t2, 0, False)
            do_tile(2 * t2 + 1, 1, False)
            return c

        lax.fori_loop(0, NTF // 2, pair_body, 0)
        do_tile(NTF, 0, True)

        emit_flush()
        upper = jnp.where(w == NW - 1, G, st[0] + 1)
        lax.fori_loop(st[1], upper, zero_seg, 0)

    tab, head = kfn(h.reshape(-1), b, e)
    return tab.reshape(G, 4, H), head.reshape(NW, 4, H)


def _merge_body(tab0_ref, head0_ref, tab1_ref, head1_ref,
                pw0_ref, pb0_ref, pw1_ref, pb1_ref,
                lng_ref, lnb_ref, f1w_ref, f1b_ref, f2w_ref, f2b_ref,
                out_ref):
    iota_g = lax.broadcasted_iota(jnp.int32, (G, 1), 0)
    lane32 = (lax.broadcasted_iota(jnp.int32, (1, H), 1) < 32)\
        .astype(jnp.float32)

    def branch(tab_ref, head_ref, pw_ref, pb_ref):
        tab = tab_ref[...]
        s_p = tab[:, 0, :]
        a_n = tab[:, 1, :]
        m_p = tab[:, 2, :]
        aux = tab[:, 3, :]
        head = head_ref[...]
        for w in range(NW):
            idf = head[w, 3, 32]
            valid = idf >= 0.0
            mask = (iota_g == idf.astype(jnp.int32)) & valid
            maskf = mask.astype(jnp.float32)
            s_p = s_p + maskf * head[w, 0, :][None, :]
            a_n = a_n + maskf * head[w, 1, :][None, :]
            m_p = jnp.maximum(m_p, jnp.where(mask, head[w, 2, :][None, :],
                                             NEG))
            aux = aux + maskf * (head[w, 3, :][None, :] * lane32)
        den = jnp.sum(aux[:, 0:16], axis=1, keepdims=True)
        cnt = jnp.sum(aux[:, 16:32], axis=1, keepdims=True)
        mean = s_p / jnp.maximum(cnt, 1.0)
        att = a_n / jnp.maximum(den, 1e-30)
        agg = jnp.concatenate([s_p, mean, m_p, att], axis=1)
        return (jnp.dot(agg, pw_ref[...], preferred_element_type=jnp.float32)
                + pb_ref[...][None, :])

    v0 = branch(tab0_ref, head0_ref, pw0_ref, pb0_ref)
    v1 = branch(tab1_ref, head1_ref, pw1_ref, pb1_ref)
    state = jnp.concatenate([v0, v1], axis=1)
    mu = jnp.mean(state, axis=-1, keepdims=True)
    var = jnp.mean((state - mu) ** 2, axis=-1, keepdims=True)
    x = (state - mu) * lax.rsqrt(var + 1e-5) * lng_ref[...][None, :] \
        + lnb_ref[...][None, :]
    x = x * jax.nn.sigmoid(x)
    x = jnp.dot(x, f1w_ref[...], preferred_element_type=jnp.float32) \
        + f1b_ref[...][None, :]
    x = x * jax.nn.sigmoid(x)
    out_ref[...] = jnp.dot(x, f2w_ref[...],
                           preferred_element_type=jnp.float32) \
        + f2b_ref[...][None, :]


def _merge(tab0, head0, tab1, head1, pW0, pb0, pW1, pb1,
           ln_g, ln_b, f1W, f1b, f2W, f2b):
    return pl.pallas_call(
        _merge_body,
        out_shape=jax.ShapeDtypeStruct((G, 8), jnp.float32),
    )(tab0, head0, tab1, head1, pW0, pb0, pW1, pb1,
      ln_g, ln_b, f1W, f1b, f2W, f2b)


def kernel(h0, b0, h1, b1, s1_W0, s1_b0, s2_W0, s2_b0, proj_W0, proj_b0,
           s1_W1, s1_b1, s2_W1, s2_b1, proj_W1, proj_b1,
           ln_g, ln_b, f1_W, f1_b, f2_W, f2_b):
    e0 = _scores(h0, s1_W0, s1_b0, s2_W0, s2_b0)
    e1 = _scores(h1, s1_W1, s1_b1, s2_W1, s2_b1)
    tab0, head0 = _sc_pool(h0, b0, e0)
    tab1, head1 = _sc_pool(h1, b1, e1)
    return _merge(tab0, head0, tab1, head1, proj_W0, proj_b0,
                  proj_W1, proj_b1, ln_g, ln_b, f1_W, f1_b, f2_W, f2_b)
